# baseline (speedup 1.0000x reference)
import jax
import jax.numpy as jnp
import numpy as np
from jax.experimental import pallas as pl
from jax.experimental.pallas import tpu as pltpu

_NEI = 1024
_MAXC = 100
_DS = 16

_TINY = np.float32(1.1754943508222875e-38)


def _threefry_bits(x1):
    ks0 = jnp.uint32(0)
    ks1 = jnp.uint32(42)
    ks2 = jnp.uint32(0 ^ 42 ^ 0x1BD11BDA)
    ks = (ks0, ks1, ks2)
    rots = ((13, 15, 26, 6), (17, 29, 16, 24))

    def rotl(v, r):
        return (v << jnp.uint32(r)) | (v >> jnp.uint32(32 - r))

    x0 = x1
    x1 = x0 ^ rotl(x1, 13)
    for r in (15, 26, 6):
        x0 = x0 + x1
        x1 = x0 ^ rotl(x1, r)
    x0 = x0 + ks[1]
    x1 = x1 + ks[2] + jnp.uint32(1)
    for i in range(1, 5):
        for r in rots[i % 2]:
            x0 = x0 + x1
            x1 = x0 ^ rotl(x1, r)
        x0 = x0 + ks[(i + 1) % 3]
        x1 = x1 + ks[(i + 2) % 3] + jnp.uint32(i + 1)
    return x0 ^ x1


def _body(yt_ref, lam_ref, theta_ref, yt1_ref, z_ref, loss_ref, colsum, acc):
    i = pl.program_id(0)
    nsteps = pl.num_programs(0)

    @pl.when(i == 0)
    def _init():
        colsum[...] = jnp.zeros_like(colsum)
        acc[0] = 0.0
        acc[1] = 0.0

    cnt = yt_ref[i]
    theta = theta_ref[0]
    recip = jnp.broadcast_to(1.0 / theta, (_DS, _NEI))

    s_io = jax.lax.broadcasted_iota(jnp.int32, (_DS, _NEI), 0)
    j_io = jax.lax.broadcasted_iota(jnp.int32, (_DS, _NEI), 1)
    idx_base = i * (_MAXC * _NEI) + s_io * _NEI + j_io + 42
    s1_io = jax.lax.broadcasted_iota(jnp.int32, (_DS, 1), 0)

    def it_body(it, z8):
        x1 = (idx_base + it * (_DS * _NEI)).astype(jnp.uint32)
        bits = _threefry_bits(x1)
        fb = (bits >> jnp.uint32(9)) | jnp.uint32(0x3F800000)
        f = jax.lax.bitcast_convert_type(fb, jnp.float32) - 1.0
        u = f + _TINY
        val = jnp.log(u) * recip
        maxv = jnp.max(val, axis=1, keepdims=True)
        msel = jnp.where(s1_io < cnt - it * _DS, maxv, 1.0)
        return z8 + jnp.where(val == msel, 1.0, 0.0)

    niter = (cnt + (_DS - 1)) // _DS
    z8 = jax.lax.fori_loop(0, niter, it_body,
                           jnp.zeros((_DS, _NEI), jnp.float32))
    z = jnp.sum(z8, axis=0, keepdims=True)
    z_ref[...] = z.reshape(1, 1, _NEI)

    theta_log = jnp.maximum(jnp.log(theta), -104.0)
    z_log = jnp.maximum(jnp.log(z), -104.0)
    obj_c = jnp.sum(z * (theta_log + 1.0 - z_log))
    et_c = (cnt.astype(jnp.float32) - jnp.sum(z)) ** 2
    acc[0] += obj_c
    acc[1] += et_c
    colsum[...] += z

    @pl.when(i == nsteps - 1)
    def _fin():
        et1 = jnp.sum((yt1_ref[...] - colsum[...]) ** 2)
        g = acc[0] - lam_ref[0, 0] * (acc[1] + et1)
        loss_ref[0, 0] = -g


def kernel(theta, yt, yt1, lam):
    n, nei = theta.shape
    theta3 = theta.reshape(n, 1, nei)
    yti = yt.astype(jnp.int32)
    yt1b = yt1.reshape(1, nei)
    lamb = jnp.asarray(lam, jnp.float32).reshape(1, 1)

    z, loss = pl.pallas_call(
        _body,
        grid=(n,),
        in_specs=[
            pl.BlockSpec(memory_space=pltpu.SMEM),
            pl.BlockSpec(memory_space=pltpu.SMEM),
            pl.BlockSpec((1, 1, nei), lambda i: (i, 0, 0)),
            pl.BlockSpec((1, nei), lambda i: (0, 0)),
        ],
        out_specs=[
            pl.BlockSpec((1, 1, nei), lambda i: (i, 0, 0)),
            pl.BlockSpec(memory_space=pltpu.SMEM),
        ],
        out_shape=[
            jax.ShapeDtypeStruct((n, 1, nei), jnp.float32),
            jax.ShapeDtypeStruct((1, 1), jnp.float32),
        ],
        scratch_shapes=[
            pltpu.VMEM((1, nei), jnp.float32),
            pltpu.SMEM((2,), jnp.float32),
        ],
    )(yti, lamb, theta3, yt1b)
    return (loss[0, 0], z.reshape(n, nei))


# speedup vs baseline: 1.3286x; 1.3286x over previous
import jax
import jax.numpy as jnp
import numpy as np
from jax.experimental import pallas as pl
from jax.experimental.pallas import tpu as pltpu

_NEI = 1024
_MAXC = 100
_R = 4
_DS = 8

_TINY = np.float32(1.1754943508222875e-38)


def _threefry_bits(x1):
    ks = (jnp.uint32(0), jnp.uint32(42), jnp.uint32(0 ^ 42 ^ 0x1BD11BDA))
    rots = ((13, 15, 26, 6), (17, 29, 16, 24))

    def rotl(v, r):
        return (v << jnp.uint32(r)) | (v >> jnp.uint32(32 - r))

    x0 = x1
    x1 = x0 ^ rotl(x1, 13)
    for r in (15, 26, 6):
        x0 = x0 + x1
        x1 = x0 ^ rotl(x1, r)
    x0 = x0 + ks[1]
    x1 = x1 + ks[2] + jnp.uint32(1)
    for i in range(1, 5):
        for r in rots[i % 2]:
            x0 = x0 + x1
            x1 = x0 ^ rotl(x1, r)
        x0 = x0 + ks[(i + 1) % 3]
        x1 = x1 + ks[(i + 2) % 3] + jnp.uint32(i + 1)
    return x0 ^ x1


def _body(yts_ref, rows_ref, lam_ref, theta_ref, yt1_ref, z_ref, loss_ref,
          colsum, acc):
    g = pl.program_id(0)
    nsteps = pl.num_programs(0)

    @pl.when(g == 0)
    def _init():
        colsum[...] = jnp.zeros_like(colsum)
        acc[0] = 0.0
        acc[1] = 0.0

    theta = theta_ref[...]
    recip = 1.0 / theta

    r_io = jax.lax.broadcasted_iota(jnp.int32, (_R, _DS, _NEI), 0)
    s_io = jax.lax.broadcasted_iota(jnp.int32, (_R, _DS, _NEI), 1)
    j_io = jax.lax.broadcasted_iota(jnp.int32, (_R, _DS, _NEI), 2)
    r1_io = jax.lax.broadcasted_iota(jnp.int32, (_R, _DS, 1), 0)
    s1_io = jax.lax.broadcasted_iota(jnp.int32, (_R, _DS, 1), 1)

    c0 = rows_ref[g * _R]
    c1 = rows_ref[g * _R + 1]
    c2 = rows_ref[g * _R + 2]
    c3 = rows_ref[g * _R + 3]
    rowid = jnp.where(r1_io == 0, c0,
            jnp.where(r1_io == 1, c1,
            jnp.where(r1_io == 2, c2, c3)))
    idx_base = rowid * (_MAXC * _NEI) + s_io * _NEI + j_io + 42

    n0 = yts_ref[g * _R]
    n1 = yts_ref[g * _R + 1]
    n2 = yts_ref[g * _R + 2]
    n3 = yts_ref[g * _R + 3]
    cnt = jnp.where(r1_io == 0, n0,
          jnp.where(r1_io == 1, n1,
          jnp.where(r1_io == 2, n2, n3)))
    maxcnt = jnp.maximum(jnp.maximum(n0, n1), jnp.maximum(n2, n3))

    def it_body(it, zacc):
        x1 = (idx_base + it * (_DS * _NEI)).astype(jnp.uint32)
        bits = _threefry_bits(x1)
        fb = (bits >> jnp.uint32(9)) | jnp.uint32(0x3F800000)
        f = jax.lax.bitcast_convert_type(fb, jnp.float32) - 1.0
        u = f + _TINY
        val = jnp.log(u) * recip
        maxv = jnp.max(val, axis=2, keepdims=True)
        msel = jnp.where(s1_io + it * _DS < cnt, maxv, 1.0)
        return zacc + jnp.where(val == msel, 1.0, 0.0)

    niter = (maxcnt + (_DS - 1)) // _DS
    zacc = jax.lax.fori_loop(0, niter, it_body,
                             jnp.zeros((_R, _DS, _NEI), jnp.float32))
    z = jnp.sum(zacc, axis=1, keepdims=True)
    z_ref[...] = z

    theta_log = jnp.maximum(jnp.log(theta), -104.0)
    z_log = jnp.maximum(jnp.log(z), -104.0)
    obj_c = jnp.sum(z * (theta_log + 1.0 - z_log))
    cntf = cnt[:, 0, :].astype(jnp.float32)
    et_c = jnp.sum((cntf - jnp.sum(z[:, 0, :], axis=1, keepdims=True)) ** 2)
    acc[0] += obj_c
    acc[1] += et_c
    colsum[...] += jnp.sum(z[:, 0, :], axis=0, keepdims=True)

    @pl.when(g == nsteps - 1)
    def _fin():
        et1 = jnp.sum((yt1_ref[...] - colsum[...]) ** 2)
        gtot = acc[0] - lam_ref[0, 0] * (acc[1] + et1)
        loss_ref[0, 0] = -gtot


def kernel(theta, yt, yt1, lam):
    n, nei = theta.shape
    yti = yt.astype(jnp.int32)
    perm = jnp.argsort(yti)
    theta_s = theta[perm].reshape(n, 1, nei)
    yts = yti[perm]
    yt1b = yt1.reshape(1, nei)
    lamb = jnp.asarray(lam, jnp.float32).reshape(1, 1)

    z_s, loss = pl.pallas_call(
        _body,
        grid=(n // _R,),
        in_specs=[
            pl.BlockSpec(memory_space=pltpu.SMEM),
            pl.BlockSpec(memory_space=pltpu.SMEM),
            pl.BlockSpec(memory_space=pltpu.SMEM),
            pl.BlockSpec((_R, 1, nei), lambda g: (g, 0, 0)),
            pl.BlockSpec((1, nei), lambda g: (0, 0)),
        ],
        out_specs=[
            pl.BlockSpec((_R, 1, nei), lambda g: (g, 0, 0)),
            pl.BlockSpec(memory_space=pltpu.SMEM),
        ],
        out_shape=[
            jax.ShapeDtypeStruct((n, 1, nei), jnp.float32),
            jax.ShapeDtypeStruct((1, 1), jnp.float32),
        ],
        scratch_shapes=[
            pltpu.VMEM((1, nei), jnp.float32),
            pltpu.SMEM((2,), jnp.float32),
        ],
    )(yts, perm.astype(jnp.int32), lamb, theta_s, yt1b)
    invperm = jnp.argsort(perm)
    return (loss[0, 0], z_s.reshape(n, nei)[invperm])


# speedup vs baseline: 1.4574x; 1.0969x over previous
import jax
import jax.numpy as jnp
import numpy as np
from jax.experimental import pallas as pl
from jax.experimental.pallas import tpu as pltpu

_NEI = 1024
_MAXC = 100
_R = 8
_DS = 8

_TINY = np.float32(1.1754943508222875e-38)


def _threefry_bits(x1):
    ks = (jnp.uint32(0), jnp.uint32(42), jnp.uint32(0 ^ 42 ^ 0x1BD11BDA))
    rots = ((13, 15, 26, 6), (17, 29, 16, 24))

    def rotl(v, r):
        return (v << jnp.uint32(r)) | (v >> jnp.uint32(32 - r))

    x0 = x1
    x1 = x0 ^ rotl(x1, 13)
    for r in (15, 26, 6):
        x0 = x0 + x1
        x1 = x0 ^ rotl(x1, r)
    x0 = x0 + ks[1]
    x1 = x1 + ks[2] + jnp.uint32(1)
    for i in range(1, 5):
        for r in rots[i % 2]:
            x0 = x0 + x1
            x1 = x0 ^ rotl(x1, r)
        x0 = x0 + ks[(i + 1) % 3]
        x1 = x1 + ks[(i + 2) % 3] + jnp.uint32(i + 1)
    return x0 ^ x1


def _body(yts_ref, rows_ref, lam_ref, theta_ref, yt1_ref, z_ref, loss_ref,
          colsum, acc):
    g = pl.program_id(0)
    nsteps = pl.num_programs(0)

    @pl.when(g == 0)
    def _init():
        colsum[...] = jnp.zeros_like(colsum)
        acc[0] = 0.0
        acc[1] = 0.0

    theta = theta_ref[...]
    recip = 1.0 / theta

    r_io = jax.lax.broadcasted_iota(jnp.int32, (_R, _DS, _NEI), 0)
    s_io = jax.lax.broadcasted_iota(jnp.int32, (_R, _DS, _NEI), 1)
    j_io = jax.lax.broadcasted_iota(jnp.int32, (_R, _DS, _NEI), 2)
    r1_io = jax.lax.broadcasted_iota(jnp.int32, (_R, _DS, 1), 0)
    s1_io = jax.lax.broadcasted_iota(jnp.int32, (_R, _DS, 1), 1)

    rs = [rows_ref[g * _R + k] for k in range(_R)]
    ns = [yts_ref[g * _R + k] for k in range(_R)]
    rowid = rs[-1]
    cnt = ns[-1]
    maxcnt = ns[-1]
    for k in reversed(range(_R - 1)):
        rowid = jnp.where(r1_io == k, rs[k], rowid)
        cnt = jnp.where(r1_io == k, ns[k], cnt)
        maxcnt = jnp.maximum(maxcnt, ns[k])
    idx_base = rowid * (_MAXC * _NEI) + s_io * _NEI + j_io + 42

    def it_body(it, zacc):
        x1 = (idx_base + it * (_DS * _NEI)).astype(jnp.uint32)
        bits = _threefry_bits(x1)
        fb = (bits >> jnp.uint32(9)) | jnp.uint32(0x3F800000)
        f = jax.lax.bitcast_convert_type(fb, jnp.float32) - 1.0
        u = f + _TINY
        val = jnp.log(u) * recip
        maxv = jnp.max(val, axis=2, keepdims=True)
        msel = jnp.where(s1_io + it * _DS < cnt, maxv, 1.0)
        return zacc + jnp.where(val == msel, 1.0, 0.0)

    niter = (maxcnt + (_DS - 1)) // _DS
    zacc = jax.lax.fori_loop(0, niter, it_body,
                             jnp.zeros((_R, _DS, _NEI), jnp.float32))
    z = jnp.sum(zacc, axis=1, keepdims=True)
    z_ref[...] = z

    theta_log = jnp.maximum(jnp.log(theta), -104.0)
    z_log = jnp.maximum(jnp.log(z), -104.0)
    obj_c = jnp.sum(z * (theta_log + 1.0 - z_log))
    cntf = cnt[:, 0, :].astype(jnp.float32)
    et_c = jnp.sum((cntf - jnp.sum(z[:, 0, :], axis=1, keepdims=True)) ** 2)
    acc[0] += obj_c
    acc[1] += et_c
    colsum[...] += jnp.sum(z[:, 0, :], axis=0, keepdims=True)

    @pl.when(g == nsteps - 1)
    def _fin():
        et1 = jnp.sum((yt1_ref[...] - colsum[...]) ** 2)
        gtot = acc[0] - lam_ref[0, 0] * (acc[1] + et1)
        loss_ref[0, 0] = -gtot


def kernel(theta, yt, yt1, lam):
    n, nei = theta.shape
    yti = yt.astype(jnp.int32)
    perm = jnp.argsort(yti)
    theta_s = theta[perm].reshape(n, 1, nei)
    yts = yti[perm]
    yt1b = yt1.reshape(1, nei)
    lamb = jnp.asarray(lam, jnp.float32).reshape(1, 1)

    z_s, loss = pl.pallas_call(
        _body,
        grid=(n // _R,),
        in_specs=[
            pl.BlockSpec(memory_space=pltpu.SMEM),
            pl.BlockSpec(memory_space=pltpu.SMEM),
            pl.BlockSpec(memory_space=pltpu.SMEM),
            pl.BlockSpec((_R, 1, nei), lambda g: (g, 0, 0)),
            pl.BlockSpec((1, nei), lambda g: (0, 0)),
        ],
        out_specs=[
            pl.BlockSpec((_R, 1, nei), lambda g: (g, 0, 0)),
            pl.BlockSpec(memory_space=pltpu.SMEM),
        ],
        out_shape=[
            jax.ShapeDtypeStruct((n, 1, nei), jnp.float32),
            jax.ShapeDtypeStruct((1, 1), jnp.float32),
        ],
        scratch_shapes=[
            pltpu.VMEM((1, nei), jnp.float32),
            pltpu.SMEM((2,), jnp.float32),
        ],
    )(yts, perm.astype(jnp.int32), lamb, theta_s, yt1b)
    invperm = jnp.argsort(perm)
    return (loss[0, 0], z_s.reshape(n, nei)[invperm])


# speedup vs baseline: 1.5051x; 1.0327x over previous
import jax
import jax.numpy as jnp
import numpy as np
from jax.experimental import pallas as pl
from jax.experimental.pallas import tpu as pltpu

_NEI = 1024
_MAXC = 100
_R = 16
_DS = 8

_TINY = np.float32(1.1754943508222875e-38)


def _threefry_bits(x1):
    ks = (jnp.uint32(0), jnp.uint32(42), jnp.uint32(0 ^ 42 ^ 0x1BD11BDA))
    rots = ((13, 15, 26, 6), (17, 29, 16, 24))

    def rotl(v, r):
        return (v << jnp.uint32(r)) | (v >> jnp.uint32(32 - r))

    x0 = x1
    x1 = x0 ^ rotl(x1, 13)
    for r in (15, 26, 6):
        x0 = x0 + x1
        x1 = x0 ^ rotl(x1, r)
    x0 = x0 + ks[1]
    x1 = x1 + ks[2] + jnp.uint32(1)
    for i in range(1, 5):
        for r in rots[i % 2]:
            x0 = x0 + x1
            x1 = x0 ^ rotl(x1, r)
        x0 = x0 + ks[(i + 1) % 3]
        x1 = x1 + ks[(i + 2) % 3] + jnp.uint32(i + 1)
    return x0 ^ x1


def _body(yts_ref, rows_ref, lam_ref, theta_ref, yt1_ref, z_ref, loss_ref,
          colsum, acc):
    g = pl.program_id(0)
    nsteps = pl.num_programs(0)

    @pl.when(g == 0)
    def _init():
        colsum[...] = jnp.zeros_like(colsum)
        acc[0] = 0.0
        acc[1] = 0.0

    theta = theta_ref[...]
    recip = 1.0 / theta

    r_io = jax.lax.broadcasted_iota(jnp.int32, (_R, _DS, _NEI), 0)
    s_io = jax.lax.broadcasted_iota(jnp.int32, (_R, _DS, _NEI), 1)
    j_io = jax.lax.broadcasted_iota(jnp.int32, (_R, _DS, _NEI), 2)
    r1_io = jax.lax.broadcasted_iota(jnp.int32, (_R, _DS, 1), 0)
    s1_io = jax.lax.broadcasted_iota(jnp.int32, (_R, _DS, 1), 1)

    rs = [rows_ref[g * _R + k] for k in range(_R)]
    ns = [yts_ref[g * _R + k] for k in range(_R)]
    rowid = rs[-1]
    cnt = ns[-1]
    maxcnt = ns[-1]
    for k in reversed(range(_R - 1)):
        rowid = jnp.where(r1_io == k, rs[k], rowid)
        cnt = jnp.where(r1_io == k, ns[k], cnt)
        maxcnt = jnp.maximum(maxcnt, ns[k])
    idx_base = rowid * (_MAXC * _NEI) + s_io * _NEI + j_io + 42

    def it_body(it, zacc):
        x1 = (idx_base + it * (_DS * _NEI)).astype(jnp.uint32)
        bits = _threefry_bits(x1)
        fb = (bits >> jnp.uint32(9)) | jnp.uint32(0x3F800000)
        f = jax.lax.bitcast_convert_type(fb, jnp.float32) - 1.0
        u = f + _TINY
        val = jnp.log(u) * recip
        maxv = jnp.max(val, axis=2, keepdims=True)
        msel = jnp.where(s1_io + it * _DS < cnt, maxv, 1.0)
        return zacc + jnp.where(val == msel, 1.0, 0.0)

    niter = (maxcnt + (_DS - 1)) // _DS
    zacc = jax.lax.fori_loop(0, niter, it_body,
                             jnp.zeros((_R, _DS, _NEI), jnp.float32))
    z = jnp.sum(zacc, axis=1, keepdims=True)
    z_ref[...] = z

    theta_log = jnp.maximum(jnp.log(theta), -104.0)
    z_log = jnp.maximum(jnp.log(z), -104.0)
    obj_c = jnp.sum(z * (theta_log + 1.0 - z_log))
    cntf = cnt[:, 0, :].astype(jnp.float32)
    et_c = jnp.sum((cntf - jnp.sum(z[:, 0, :], axis=1, keepdims=True)) ** 2)
    acc[0] += obj_c
    acc[1] += et_c
    colsum[...] += jnp.sum(z[:, 0, :], axis=0, keepdims=True)

    @pl.when(g == nsteps - 1)
    def _fin():
        et1 = jnp.sum((yt1_ref[...] - colsum[...]) ** 2)
        gtot = acc[0] - lam_ref[0, 0] * (acc[1] + et1)
        loss_ref[0, 0] = -gtot


def kernel(theta, yt, yt1, lam):
    n, nei = theta.shape
    yti = yt.astype(jnp.int32)
    perm = jnp.argsort(yti)
    theta_s = theta[perm].reshape(n, 1, nei)
    yts = yti[perm]
    yt1b = yt1.reshape(1, nei)
    lamb = jnp.asarray(lam, jnp.float32).reshape(1, 1)

    z_s, loss = pl.pallas_call(
        _body,
        grid=(n // _R,),
        in_specs=[
            pl.BlockSpec(memory_space=pltpu.SMEM),
            pl.BlockSpec(memory_space=pltpu.SMEM),
            pl.BlockSpec(memory_space=pltpu.SMEM),
            pl.BlockSpec((_R, 1, nei), lambda g: (g, 0, 0)),
            pl.BlockSpec((1, nei), lambda g: (0, 0)),
        ],
        out_specs=[
            pl.BlockSpec((_R, 1, nei), lambda g: (g, 0, 0)),
            pl.BlockSpec(memory_space=pltpu.SMEM),
        ],
        out_shape=[
            jax.ShapeDtypeStruct((n, 1, nei), jnp.float32),
            jax.ShapeDtypeStruct((1, 1), jnp.float32),
        ],
        scratch_shapes=[
            pltpu.VMEM((1, nei), jnp.float32),
            pltpu.SMEM((2,), jnp.float32),
        ],
    )(yts, perm.astype(jnp.int32), lamb, theta_s, yt1b)
    invperm = jnp.argsort(perm)
    return (loss[0, 0], z_s.reshape(n, nei)[invperm])
